# Initial kernel scaffold; baseline (speedup 1.0000x reference)
import jax
import jax.numpy as jnp
from jax.experimental import pallas as pl


def kernel(x):
    raise NotImplementedError("write your pallas kernel here")


# speedup vs baseline: 4.4715x; 4.4715x over previous
import jax
import jax.numpy as jnp
from jax.experimental import pallas as pl
from jax.experimental.pallas import tpu as pltpu

_BLK = 256


def _body(x_ref, o_ref, carry_ref, tril_ref):
    t = pl.program_id(1)

    @pl.when(t == 0)
    def _():
        carry_ref[...] = jnp.zeros_like(carry_ref)
        i = jax.lax.broadcasted_iota(jnp.int32, (_BLK, _BLK), 0)
        j = jax.lax.broadcasted_iota(jnp.int32, (_BLK, _BLK), 1)
        tril_ref[...] = jnp.where(j <= i, 1.0, 0.0).astype(jnp.bfloat16)

    xb = x_ref[0].astype(jnp.bfloat16)
    part = jnp.dot(
        tril_ref[...], xb, preferred_element_type=jnp.float32
    )
    tot = part + carry_ref[...]

    base = (t * _BLK + 1).astype(jnp.float32)
    pos = (
        jax.lax.broadcasted_iota(jnp.int32, (_BLK, 128), 0).astype(jnp.float32)
        + base
    )
    inv = pltpu.repeat(1.0 / pos, o_ref.shape[2] // 128, axis=1)

    o_ref[0] = tot * inv
    carry_ref[...] = tot[_BLK - 1 : _BLK, :]


def kernel(x):
    B, T, C = x.shape
    grid = (B, T // _BLK)
    return pl.pallas_call(
        _body,
        out_shape=jax.ShapeDtypeStruct((B, T, C), x.dtype),
        grid=grid,
        in_specs=[
            pl.BlockSpec((1, _BLK, C), lambda b, t: (b, t, 0)),
        ],
        out_specs=pl.BlockSpec((1, _BLK, C), lambda b, t: (b, t, 0)),
        scratch_shapes=[
            pltpu.VMEM((1, C), jnp.float32),
            pltpu.VMEM((_BLK, _BLK), jnp.bfloat16),
        ],
        compiler_params=pltpu.CompilerParams(
            dimension_semantics=("parallel", "arbitrary"),
        ),
        name="causal_mean_pool",
    )(x)


# speedup vs baseline: 7.2088x; 1.6122x over previous
import jax
import jax.numpy as jnp
from jax.experimental import pallas as pl
from jax.experimental.pallas import tpu as pltpu

_BLK = 1024
_SUB = 256


def _body(x_ref, o_ref, carry_ref, tril_ref):
    t = pl.program_id(1)

    @pl.when(t == 0)
    def _():
        carry_ref[...] = jnp.zeros_like(carry_ref)
        i = jax.lax.broadcasted_iota(jnp.int32, (_SUB, _SUB), 0)
        j = jax.lax.broadcasted_iota(jnp.int32, (_SUB, _SUB), 1)
        tril_ref[...] = jnp.where(j <= i, 1.0, 0.0).astype(jnp.bfloat16)

    tril = tril_ref[...]
    carry = carry_ref[...]
    lanes = o_ref.shape[2] // 128
    for s in range(_BLK // _SUB):
        xb = x_ref[0, s * _SUB : (s + 1) * _SUB, :].astype(jnp.bfloat16)
        part = jnp.dot(tril, xb, preferred_element_type=jnp.float32)
        tot = part + carry
        carry = tot[_SUB - 1 : _SUB, :]

        base = (t * _BLK + s * _SUB + 1).astype(jnp.float32)
        pos = (
            jax.lax.broadcasted_iota(jnp.int32, (_SUB, 128), 0).astype(
                jnp.float32
            )
            + base
        )
        inv = pltpu.repeat(1.0 / pos, lanes, axis=1)
        o_ref[0, s * _SUB : (s + 1) * _SUB, :] = tot * inv
    carry_ref[...] = carry


def kernel(x):
    B, T, C = x.shape
    grid = (B, T // _BLK)
    return pl.pallas_call(
        _body,
        out_shape=jax.ShapeDtypeStruct((B, T, C), x.dtype),
        grid=grid,
        in_specs=[
            pl.BlockSpec((1, _BLK, C), lambda b, t: (b, t, 0)),
        ],
        out_specs=pl.BlockSpec((1, _BLK, C), lambda b, t: (b, t, 0)),
        scratch_shapes=[
            pltpu.VMEM((1, C), jnp.float32),
            pltpu.VMEM((_SUB, _SUB), jnp.bfloat16),
        ],
        compiler_params=pltpu.CompilerParams(
            dimension_semantics=("parallel", "arbitrary"),
        ),
        name="causal_mean_pool",
    )(x)


# speedup vs baseline: 7.4115x; 1.0281x over previous
import jax
import jax.numpy as jnp
from jax.experimental import pallas as pl
from jax.experimental.pallas import tpu as pltpu

_BLK = 2048
_SUB = 256


def _body(x_ref, o_ref, carry_ref, tril_ref):
    t = pl.program_id(1)

    @pl.when(t == 0)
    def _():
        carry_ref[...] = jnp.zeros_like(carry_ref)
        i = jax.lax.broadcasted_iota(jnp.int32, (_SUB, _SUB), 0)
        j = jax.lax.broadcasted_iota(jnp.int32, (_SUB, _SUB), 1)
        tril_ref[...] = jnp.where(j <= i, 1.0, 0.0).astype(jnp.bfloat16)

    tril = tril_ref[...]
    carry = carry_ref[...]
    lanes = o_ref.shape[2] // 128
    for s in range(_BLK // _SUB):
        xb = x_ref[0, s * _SUB : (s + 1) * _SUB, :].astype(jnp.bfloat16)
        part = jnp.dot(tril, xb, preferred_element_type=jnp.float32)
        tot = part + carry
        carry = tot[_SUB - 1 : _SUB, :]

        base = (t * _BLK + s * _SUB + 1).astype(jnp.float32)
        pos = (
            jax.lax.broadcasted_iota(jnp.int32, (_SUB, 128), 0).astype(
                jnp.float32
            )
            + base
        )
        inv = pltpu.repeat(1.0 / pos, lanes, axis=1)
        o_ref[0, s * _SUB : (s + 1) * _SUB, :] = tot * inv
    carry_ref[...] = carry


def kernel(x):
    B, T, C = x.shape
    grid = (B, T // _BLK)
    return pl.pallas_call(
        _body,
        out_shape=jax.ShapeDtypeStruct((B, T, C), x.dtype),
        grid=grid,
        in_specs=[
            pl.BlockSpec((1, _BLK, C), lambda b, t: (b, t, 0)),
        ],
        out_specs=pl.BlockSpec((1, _BLK, C), lambda b, t: (b, t, 0)),
        scratch_shapes=[
            pltpu.VMEM((1, C), jnp.float32),
            pltpu.VMEM((_SUB, _SUB), jnp.bfloat16),
        ],
        compiler_params=pltpu.CompilerParams(
            dimension_semantics=("parallel", "arbitrary"),
            vmem_limit_bytes=48 * 1024 * 1024,
        ),
        name="causal_mean_pool",
    )(x)
